# Initial kernel scaffold; baseline (speedup 1.0000x reference)
import jax
import jax.numpy as jnp
from jax.experimental import pallas as pl


def kernel(x, edge_index, edge_attr, freqband_order, Wn, bn, We, be, W1, b1, g1, bt1, W2, b2, g2, bt2):
    raise NotImplementedError("write your pallas kernel here")


# speedup vs baseline: 3.5698x; 3.5698x over previous
import functools

import jax
import jax.numpy as jnp
from jax import lax
from jax.experimental import pallas as pl
from jax.experimental.pallas import tpu as pltpu
from jax.experimental.pallas import tpu_sc as plsc

N = 10000
E = 320000
D = 128
DE = 16
NSEG = 1000

NP = 10240
EP = 323584
NC = 2
NS = 16
NW = NC * NS
EPW = EP // NW
CHUNK = 128
NCH = EPW // CHUNK
RPT = NP // NS

SEGACC = 1024
SEGCH = 64
NSEGCH = (NP // NW) // SEGCH

_MESH = plsc.VectorSubcoreMesh(core_axis_name="c", subcore_axis_name="s")


def _zero_f32(ref, rows, cols):
    @pl.loop(0, rows)
    def _(r):
        @pl.loop(0, cols, step=16)
        def _(c):
            ref[r, pl.ds(c, 16)] = jnp.zeros((16,), jnp.float32)


def _edge_pass(h, src_i, dst_i):
    out = jax.ShapeDtypeStruct((NC, NP, D), jnp.float32)
    scratch = [
        pltpu.VMEM((NCH, CHUNK), jnp.int32),
        pltpu.VMEM((NCH, CHUNK), jnp.int32),
        pltpu.VMEM((CHUNK, D), jnp.float32),
        pltpu.VMEM_SHARED((NP, D), jnp.float32),
    ]

    def body(h_hbm, src_hbm, dst_hbm, hp_hbm, src_v, dst_v, rows_v, acc_h):
        cid = lax.axis_index("c")
        sid = lax.axis_index("s")
        w = cid * NS + sid
        base = sid * RPT

        _zero_f32(rows_v, CHUNK, D)
        for k in range(RPT // CHUNK):
            pltpu.sync_copy(rows_v, acc_h.at[pl.ds(base + k * CHUNK, CHUNK)])

        pltpu.sync_copy(src_hbm.at[w], src_v)
        pltpu.sync_copy(dst_hbm.at[w], dst_v)
        plsc.subcore_barrier()

        @pl.loop(0, NCH)
        def _(j):
            pltpu.sync_copy(h_hbm.at[src_v.at[j]], rows_v)
            pltpu.sync_copy(rows_v, acc_h.at[dst_v.at[j]], add=True)

        plsc.subcore_barrier()
        pltpu.sync_copy(acc_h.at[pl.ds(base, RPT)],
                        hp_hbm.at[cid, pl.ds(base, RPT)])

    f = pl.kernel(body, out_type=out, mesh=_MESH, scratch_types=scratch)
    res = f(h, src_i, dst_i)
    if isinstance(res, (tuple, list)):
        (res,) = res
    return res


def _ea_pass(dst_i, ea):
    out = jax.ShapeDtypeStruct((NC, NP, D), jnp.float32)
    scratch = [
        pltpu.VMEM((NCH, CHUNK), jnp.int32),
        pltpu.VMEM((CHUNK, DE), jnp.float32),
        pltpu.VMEM((CHUNK, D), jnp.float32),
        pltpu.VMEM_SHARED((NP, D), jnp.float32),
    ]

    def body(dst_hbm, ea_hbm, eap_hbm, dst_v, ea_v, rows_v, acc):
        cid = lax.axis_index("c")
        sid = lax.axis_index("s")
        w = cid * NS + sid
        base = sid * RPT

        _zero_f32(rows_v, CHUNK, D)
        for k in range(RPT // CHUNK):
            pltpu.sync_copy(rows_v, acc.at[pl.ds(base + k * CHUNK, CHUNK)])

        one0 = jnp.where(lax.iota(jnp.int32, 16) == 0, 1.0, 0.0).astype(
            jnp.float32)

        @pl.loop(0, CHUNK)
        def _(r):
            rows_v[r, pl.ds(16, 16)] = one0

        pltpu.sync_copy(dst_hbm.at[w], dst_v)
        plsc.subcore_barrier()

        ebase = w * EPW

        @pl.loop(0, NCH)
        def _(j):
            pltpu.sync_copy(ea_hbm.at[pl.ds(ebase + j * CHUNK, CHUNK)], ea_v)

            @pl.loop(0, CHUNK)
            def _(r):
                rows_v[r, pl.ds(0, 16)] = ea_v[r, pl.ds(0, 16)]

            pltpu.sync_copy(rows_v, acc.at[dst_v.at[j]], add=True)

        plsc.subcore_barrier()
        pltpu.sync_copy(acc.at[pl.ds(base, RPT)],
                        eap_hbm.at[cid, pl.ds(base, RPT)])

    f = pl.kernel(body, out_type=out, mesh=_MESH, scratch_types=scratch)
    res = f(dst_i, ea)
    if isinstance(res, (tuple, list)):
        (res,) = res
    return res


def _seg_pass(h, seg_i):
    out = jax.ShapeDtypeStruct((NC, 2 * SEGACC, D), jnp.float32)
    scratch = [
        pltpu.VMEM((2 * NSEGCH, SEGCH), jnp.int32),
        pltpu.VMEM((SEGCH, D), jnp.float32),
        pltpu.VMEM((SEGCH, D), jnp.float32),
        pltpu.VMEM_SHARED((2 * SEGACC, D), jnp.float32),
    ]
    rpt = 2 * SEGACC // NS

    def body(h_hbm, seg_hbm, sp_hbm, seg_v, rows_v, ones_v, acc):
        cid = lax.axis_index("c")
        sid = lax.axis_index("s")
        w = cid * NS + sid
        base = sid * rpt

        _zero_f32(rows_v, SEGCH, D)

        ones = jnp.ones((16,), jnp.float32)

        @pl.loop(0, SEGCH)
        def _(r):
            @pl.loop(0, D, step=16)
            def _(c):
                ones_v[r, pl.ds(c, 16)] = ones

        for k in range(rpt // SEGCH):
            pltpu.sync_copy(rows_v, acc.at[pl.ds(base + k * SEGCH, SEGCH)])
        pltpu.sync_copy(seg_hbm.at[w], seg_v)
        plsc.subcore_barrier()

        rbase = w * (NP // NW)

        @pl.loop(0, NSEGCH)
        def _(j):
            pltpu.sync_copy(h_hbm.at[pl.ds(rbase + j * SEGCH, SEGCH)], rows_v)
            pltpu.sync_copy(rows_v, acc.at[seg_v.at[j]], add=True)
            pltpu.sync_copy(ones_v, acc.at[seg_v.at[j + NSEGCH]], add=True)

        plsc.subcore_barrier()
        pltpu.sync_copy(acc.at[pl.ds(base, rpt)],
                        sp_hbm.at[cid, pl.ds(base, rpt)])

    f = pl.kernel(body, out_type=out, mesh=_MESH, scratch_types=scratch)
    res = f(h, seg_i)
    if isinstance(res, (tuple, list)):
        (res,) = res
    return res


_BLK = 512
_GRID = NP // _BLK


def _embed_body(x_ref, w_ref, b_ref, o_ref):
    o_ref[...] = (jnp.dot(x_ref[...], w_ref[...],
                          preferred_element_type=jnp.float32) + b_ref[...])


def _embed(x, w, b):
    return pl.pallas_call(
        _embed_body,
        grid=(_GRID,),
        in_specs=[
            pl.BlockSpec((_BLK, D), lambda i: (i, 0)),
            pl.BlockSpec((D, D), lambda i: (0, 0)),
            pl.BlockSpec((1, D), lambda i: (0, 0)),
        ],
        out_specs=pl.BlockSpec((_BLK, D), lambda i: (i, 0)),
        out_shape=jax.ShapeDtypeStruct((NP, D), jnp.float32),
    )(x, w, b.reshape(1, D))


def _layer_body(hp_ref, ec_ref, we_ref, be_ref, w_ref, b_ref,
                g_ref, bt_ref, o_ref):
    p = hp_ref[0] + hp_ref[1]
    ec = ec_ref[0] + ec_ref[1]
    eas = ec[:, 0:DE]
    cnt = ec[:, DE:DE + 1]
    eagg = (jnp.dot(eas, we_ref[...], preferred_element_type=jnp.float32)
            + cnt * be_ref[...])
    agg = (p + eagg) / (cnt + 1.0)
    t = (jnp.dot(agg, w_ref[...], preferred_element_type=jnp.float32)
         + b_ref[...])
    u = jax.nn.gelu(t)
    mu = jnp.mean(u, axis=-1, keepdims=True)
    var = jnp.mean((u - mu) * (u - mu), axis=-1, keepdims=True)
    o_ref[...] = (u - mu) / jnp.sqrt(var + 1e-5) * g_ref[...] + bt_ref[...]


def _layer(hp, eacnt, we, be, w, b, g, bt):
    return pl.pallas_call(
        _layer_body,
        grid=(_GRID,),
        in_specs=[
            pl.BlockSpec((NC, _BLK, D), lambda i: (0, i, 0)),
            pl.BlockSpec((NC, _BLK, D), lambda i: (0, i, 0)),
            pl.BlockSpec((DE, D), lambda i: (0, 0)),
            pl.BlockSpec((1, D), lambda i: (0, 0)),
            pl.BlockSpec((D, D), lambda i: (0, 0)),
            pl.BlockSpec((1, D), lambda i: (0, 0)),
            pl.BlockSpec((1, D), lambda i: (0, 0)),
            pl.BlockSpec((1, D), lambda i: (0, 0)),
        ],
        out_specs=pl.BlockSpec((_BLK, D), lambda i: (i, 0)),
        out_shape=jax.ShapeDtypeStruct((NP, D), jnp.float32),
    )(hp, eacnt, we, be.reshape(1, D), w, b.reshape(1, D),
      g.reshape(1, D), bt.reshape(1, D))


def _fb_body(sp_ref, o_ref):
    s = sp_ref[0] + sp_ref[1]
    cnt = s[SEGACC:SEGACC + NSEG, 0:1]
    o_ref[...] = s[:NSEG] / jnp.maximum(cnt, 1.0)


def _fb(sp):
    return pl.pallas_call(
        _fb_body,
        grid=(1,),
        in_specs=[
            pl.BlockSpec((NC, 2 * SEGACC, D), lambda i: (0, 0, 0)),
        ],
        out_specs=pl.BlockSpec((NSEG, D), lambda i: (0, 0)),
        out_shape=jax.ShapeDtypeStruct((NSEG, D), jnp.float32),
    )(sp)


def kernel(x, edge_index, edge_attr, freqband_order, Wn, bn, We, be,
           W1, b1, g1, bt1, W2, b2, g2, bt2):
    xp = jnp.concatenate(
        [x, jnp.zeros((NP - N, D), jnp.float32)], axis=0)
    src = jnp.concatenate(
        [edge_index[0], jnp.zeros((EP - E,), jnp.int32)]).reshape(
            NW, NCH, CHUNK)
    dst = jnp.concatenate(
        [edge_index[1], jnp.full((EP - E,), NP - 1, jnp.int32)]).reshape(
            NW, NCH, CHUNK)
    eap = jnp.concatenate(
        [edge_attr, jnp.zeros((EP - E, DE), jnp.float32)], axis=0)
    seg1 = jnp.concatenate(
        [freqband_order[:, 0], jnp.full((NP - N,), NSEG, jnp.int32)]).reshape(
            NW, NSEGCH, SEGCH)
    seg = jnp.concatenate([seg1, seg1 + SEGACC], axis=1)

    h0 = _embed(xp, Wn, bn)
    eacnt = _ea_pass(dst, eap)
    hp1 = _edge_pass(h0, src, dst)
    h1 = _layer(hp1, eacnt, We, be, W1, b1, g1, bt1)
    hp2 = _edge_pass(h1, src, dst)
    h2 = _layer(hp2, eacnt, We, be, W2, b2, g2, bt2)
    sp = _seg_pass(h2, seg)
    fb = _fb(sp)
    return fb.reshape(NSEG, 1, D)
